# baseline (speedup 1.0000x reference)
import dataclasses
import functools

import jax
import jax.numpy as jnp
from jax import lax
from jax.experimental import pallas as pl
from jax.experimental.pallas import tpu as pltpu
from jax.experimental.pallas import tpu_sc as plsc

N_USERS = 5000
N_ITEMS = 5000
N = N_USERS + N_ITEMS
D = 128
NP = 10112
E = 320000
N_LAYERS = 3
BATCH = 4096

NC, NS = 2, 16
NW = NC * NS
CH = 128
NCH = 80
HLF = NCH // 2
EW = NCH * CH
EP = NW * EW
RT = NP // NS

_mesh = plsc.VectorSubcoreMesh(core_axis_name="c", subcore_axis_name="s")

_sc_params = pltpu.CompilerParams()
if "needs_layout_passes" in pltpu.CompilerParams.__dataclass_fields__:
    _sc_params = dataclasses.replace(_sc_params, needs_layout_passes=False)


def _deg_body(row_hbm, part_hbm, hist, slab):
    cid = lax.axis_index("c")
    sid = lax.axis_index("s")
    wid = cid * NS + sid

    @pl.loop(0, NP, step=16)
    def _(i):
        hist[pl.ds(i, 16)] = jnp.zeros((16,), jnp.float32)

    pltpu.sync_copy(row_hbm.at[wid], slab)
    ones = jnp.ones((16,), jnp.float32)

    @pl.loop(0, NCH)
    def _(c):
        @pl.loop(0, CH, step=16)
        def _(j):
            plsc.addupdate_scatter(hist, [slab[c, pl.ds(j, 16)]], ones)

    pltpu.sync_copy(hist, part_hbm.at[wid])


_deg_call = pl.kernel(
    _deg_body,
    out_type=jax.ShapeDtypeStruct((NW, NP), jnp.float32),
    mesh=_mesh,
    scratch_types=[
        pltpu.VMEM((NP,), jnp.float32),
        pltpu.VMEM((NCH, CH), jnp.int32),
    ],
    compiler_params=_sc_params,
)


def _prop_body(x_hbm, row_hbm, col_hbm, zeros_hbm, part_hbm,
               accum, rows0, rows1, ridx, cidx, semg0, semg1, sems0, sems1):
    cid = lax.axis_index("c")
    sid = lax.axis_index("s")
    wid = cid * NS + sid

    pltpu.sync_copy(zeros_hbm.at[pl.ds(sid * RT, RT)],
                    accum.at[pl.ds(sid * RT, RT)])
    plsc.subcore_barrier()

    for h in range(2):
        pltpu.sync_copy(row_hbm.at[wid, pl.ds(h * HLF, HLF)], ridx)
        pltpu.sync_copy(col_hbm.at[wid, pl.ds(h * HLF, HLF)], cidx)
        pltpu.async_copy(x_hbm.at[ridx.at[0]], rows0, semg0)
        pltpu.async_copy(x_hbm.at[ridx.at[1]], rows1, semg1)

        @pl.loop(0, HLF, step=2)
        def _(c):
            pltpu.make_async_copy(x_hbm.at[ridx.at[c]], rows0, semg0).wait()
            pltpu.async_copy(rows0, accum.at[cidx.at[c]], sems0, add=True)

            pltpu.make_async_copy(x_hbm.at[ridx.at[c + 1]], rows1, semg1).wait()
            pltpu.async_copy(rows1, accum.at[cidx.at[c + 1]], sems1, add=True)

            pltpu.make_async_copy(rows0, accum.at[cidx.at[c]], sems0).wait()

            @pl.when(c + 2 < HLF)
            def _():
                pltpu.async_copy(x_hbm.at[ridx.at[c + 2]], rows0, semg0)

            pltpu.make_async_copy(rows1, accum.at[cidx.at[c + 1]], sems1).wait()

            @pl.when(c + 3 < HLF)
            def _():
                pltpu.async_copy(x_hbm.at[ridx.at[c + 3]], rows1, semg1)

    plsc.subcore_barrier()
    pltpu.sync_copy(accum.at[pl.ds(sid * RT, RT)],
                    part_hbm.at[cid, pl.ds(sid * RT, RT)])


_prop_call = pl.kernel(
    _prop_body,
    out_type=jax.ShapeDtypeStruct((NC, NP, D), jnp.float32),
    mesh=_mesh,
    scratch_types=[
        pltpu.VMEM_SHARED((NP, D), jnp.float32),
        pltpu.VMEM((CH, D), jnp.float32),
        pltpu.VMEM((CH, D), jnp.float32),
        pltpu.VMEM((HLF, CH), jnp.int32),
        pltpu.VMEM((HLF, CH), jnp.int32),
        pltpu.SemaphoreType.DMA,
        pltpu.SemaphoreType.DMA,
        pltpu.SemaphoreType.DMA,
        pltpu.SemaphoreType.DMA,
    ],
)


def _bgather_body(s_hbm, gidx_hbm, out_hbm, idxb, rows):
    cid = lax.axis_index("c")
    sid = lax.axis_index("s")
    wid = cid * NS + sid

    @pl.loop(0, 2)
    def _(c):
        base = wid * 256 + c * CH
        pltpu.sync_copy(gidx_hbm.at[pl.ds(base, CH)], idxb)
        pltpu.sync_copy(s_hbm.at[idxb], rows)
        pltpu.sync_copy(rows, out_hbm.at[pl.ds(base, CH)])


_bgather_call = pl.kernel(
    _bgather_body,
    out_type=jax.ShapeDtypeStruct((2 * BATCH, D), jnp.float32),
    mesh=_mesh,
    scratch_types=[
        pltpu.VMEM((CH,), jnp.int32),
        pltpu.VMEM((CH, D), jnp.float32),
    ],
)


def _prep_kernel(part_ref, e0_ref, dinv_ref, x0_ref):
    deg = jnp.sum(part_ref[...], axis=0)
    dinv = jnp.where(deg > 0, lax.rsqrt(deg), 0.0)
    db = jnp.broadcast_to(dinv[:, None], (NP, D))
    dinv_ref[...] = db
    x0_ref[...] = e0_ref[...] * db


def _combine_kernel(part_ref, dinv_ref, s_ref, x_ref, snew_ref):
    merged = part_ref[0] + part_ref[1]
    db = dinv_ref[...]
    e = db * merged
    x_ref[...] = db * e
    snew_ref[...] = s_ref[...] + e


def _dot_kernel(rows_ref, out_ref):
    u = rows_ref[0:BATCH, :]
    v = rows_ref[BATCH:2 * BATCH, :]
    s = jnp.sum(u * v, axis=1) * (1.0 / 16.0)
    out_ref[...] = s.reshape(32, BATCH // 32)


_prep_call = pl.pallas_call(
    _prep_kernel,
    out_shape=(
        jax.ShapeDtypeStruct((NP, D), jnp.float32),
        jax.ShapeDtypeStruct((NP, D), jnp.float32),
    ),
)

_combine_call = pl.pallas_call(
    _combine_kernel,
    out_shape=(
        jax.ShapeDtypeStruct((NP, D), jnp.float32),
        jax.ShapeDtypeStruct((NP, D), jnp.float32),
    ),
)

_dot_call = pl.pallas_call(
    _dot_kernel,
    out_shape=jax.ShapeDtypeStruct((32, BATCH // 32), jnp.float32),
)


@jax.jit
def kernel(users, items, edge_index, user_table, item_table):
    row = edge_index[0].astype(jnp.int32)
    col = edge_index[1].astype(jnp.int32)
    pad = N + jnp.arange(EP - E, dtype=jnp.int32) % (NP - N)
    row_p = jnp.concatenate([row, pad]).reshape(NW, NCH, CH)
    col_p = jnp.concatenate([col, pad]).reshape(NW, NCH, CH)

    e0 = jnp.concatenate([user_table, item_table], axis=0)
    e0 = jnp.pad(e0, ((0, NP - N), (0, 0)))
    zeros = jnp.zeros((NP, D), jnp.float32)

    deg_part = _deg_call(row_p)
    dinv_b, x = _prep_call(deg_part, e0)

    s = e0
    for _ in range(N_LAYERS):
        part = _prop_call(x, row_p, col_p, zeros)
        x, s = _combine_call(part, dinv_b, s)

    gidx = jnp.concatenate([users.astype(jnp.int32),
                            items.astype(jnp.int32) + N_USERS])
    rows = _bgather_call(s, gidx)
    scores = _dot_call(rows)
    return scores.reshape(BATCH)


# speedup vs baseline: 1.2573x; 1.2573x over previous
import dataclasses
import functools

import jax
import jax.numpy as jnp
from jax import lax
from jax.experimental import pallas as pl
from jax.experimental.pallas import tpu as pltpu
from jax.experimental.pallas import tpu_sc as plsc

N_USERS = 5000
N_ITEMS = 5000
N = N_USERS + N_ITEMS
D = 128
NP = 10112
E = 320000
N_LAYERS = 3
BATCH = 4096

NC, NS = 2, 16
NW = NC * NS
CH = 128
NCH = 80
HLF = NCH // 2
EW = NCH * CH
EP = NW * EW
RT = NP // NS

_mesh = plsc.VectorSubcoreMesh(core_axis_name="c", subcore_axis_name="s")

_sc_params = pltpu.CompilerParams()
if "needs_layout_passes" in pltpu.CompilerParams.__dataclass_fields__:
    _sc_params = dataclasses.replace(_sc_params, needs_layout_passes=False)


def _deg_body(row_hbm, part_hbm, hist, slab):
    cid = lax.axis_index("c")
    sid = lax.axis_index("s")
    wid = cid * NS + sid

    @pl.loop(0, NP, step=16)
    def _(i):
        hist[pl.ds(i, 16)] = jnp.zeros((16,), jnp.float32)

    pltpu.sync_copy(row_hbm.at[wid], slab)
    ones = jnp.ones((16,), jnp.float32)

    @pl.loop(0, NCH)
    def _(c):
        @pl.loop(0, CH, step=16)
        def _(j):
            plsc.addupdate_scatter(hist, [slab[c, pl.ds(j, 16)]], ones)

    pltpu.sync_copy(hist, part_hbm.at[wid])


_deg_call = pl.kernel(
    _deg_body,
    out_type=jax.ShapeDtypeStruct((NW, NP), jnp.float32),
    mesh=_mesh,
    scratch_types=[
        pltpu.VMEM((NP,), jnp.float32),
        pltpu.VMEM((NCH, CH), jnp.int32),
    ],
    compiler_params=_sc_params,
)


def _prop_body(x_hbm, row_hbm, col_hbm, zeros_hbm, part_hbm,
               accum, rows0, rows1, ridx, cidx, semg0, semg1, sems0, sems1):
    cid = lax.axis_index("c")
    sid = lax.axis_index("s")
    wid = cid * NS + sid

    pltpu.sync_copy(zeros_hbm.at[pl.ds(sid * RT, RT)],
                    accum.at[pl.ds(sid * RT, RT)])
    plsc.subcore_barrier()

    for h in range(2):
        pltpu.sync_copy(row_hbm.at[wid, pl.ds(h * HLF, HLF)], ridx)
        pltpu.sync_copy(col_hbm.at[wid, pl.ds(h * HLF, HLF)], cidx)
        pltpu.async_copy(x_hbm.at[ridx.at[0]], rows0, semg0)
        pltpu.async_copy(x_hbm.at[ridx.at[1]], rows1, semg1)

        @pl.loop(0, HLF, step=2)
        def _(c):
            pltpu.make_async_copy(x_hbm.at[ridx.at[c]], rows0, semg0).wait()
            pltpu.sync_copy(rows0, accum.at[cidx.at[c]], add=True)

            @pl.when(c + 2 < HLF)
            def _():
                pltpu.async_copy(x_hbm.at[ridx.at[c + 2]], rows0, semg0)

            pltpu.make_async_copy(x_hbm.at[ridx.at[c + 1]], rows1, semg1).wait()
            pltpu.sync_copy(rows1, accum.at[cidx.at[c + 1]], add=True)

            @pl.when(c + 3 < HLF)
            def _():
                pltpu.async_copy(x_hbm.at[ridx.at[c + 3]], rows1, semg1)

    plsc.subcore_barrier()
    pltpu.sync_copy(accum.at[pl.ds(sid * RT, RT)],
                    part_hbm.at[cid, pl.ds(sid * RT, RT)])


_prop_call = pl.kernel(
    _prop_body,
    out_type=jax.ShapeDtypeStruct((NC, NP, D), jnp.float32),
    mesh=_mesh,
    scratch_types=[
        pltpu.VMEM_SHARED((NP, D), jnp.float32),
        pltpu.VMEM((CH, D), jnp.float32),
        pltpu.VMEM((CH, D), jnp.float32),
        pltpu.VMEM((HLF, CH), jnp.int32),
        pltpu.VMEM((HLF, CH), jnp.int32),
        pltpu.SemaphoreType.DMA,
        pltpu.SemaphoreType.DMA,
        pltpu.SemaphoreType.DMA,
        pltpu.SemaphoreType.DMA,
    ],
)


def _bgather_body(s_hbm, gidx_hbm, out_hbm, idxb, rows):
    cid = lax.axis_index("c")
    sid = lax.axis_index("s")
    wid = cid * NS + sid

    @pl.loop(0, 2)
    def _(c):
        base = wid * 256 + c * CH
        pltpu.sync_copy(gidx_hbm.at[pl.ds(base, CH)], idxb)
        pltpu.sync_copy(s_hbm.at[idxb], rows)
        pltpu.sync_copy(rows, out_hbm.at[pl.ds(base, CH)])


_bgather_call = pl.kernel(
    _bgather_body,
    out_type=jax.ShapeDtypeStruct((2 * BATCH, D), jnp.float32),
    mesh=_mesh,
    scratch_types=[
        pltpu.VMEM((CH,), jnp.int32),
        pltpu.VMEM((CH, D), jnp.float32),
    ],
)


def _prep_kernel(part_ref, e0_ref, dinv_ref, x0_ref):
    deg = jnp.sum(part_ref[...], axis=0)
    dinv = jnp.where(deg > 0, lax.rsqrt(deg), 0.0)
    db = jnp.broadcast_to(dinv[:, None], (NP, D))
    dinv_ref[...] = db
    x0_ref[...] = e0_ref[...] * db


def _combine_kernel(part_ref, dinv_ref, s_ref, x_ref, snew_ref):
    merged = part_ref[0] + part_ref[1]
    db = dinv_ref[...]
    e = db * merged
    x_ref[...] = db * e
    snew_ref[...] = s_ref[...] + e


def _dot_kernel(rows_ref, out_ref):
    u = rows_ref[0:BATCH, :]
    v = rows_ref[BATCH:2 * BATCH, :]
    s = jnp.sum(u * v, axis=1) * (1.0 / 16.0)
    out_ref[...] = s.reshape(32, BATCH // 32)


_prep_call = pl.pallas_call(
    _prep_kernel,
    out_shape=(
        jax.ShapeDtypeStruct((NP, D), jnp.float32),
        jax.ShapeDtypeStruct((NP, D), jnp.float32),
    ),
)

_combine_call = pl.pallas_call(
    _combine_kernel,
    out_shape=(
        jax.ShapeDtypeStruct((NP, D), jnp.float32),
        jax.ShapeDtypeStruct((NP, D), jnp.float32),
    ),
)

_dot_call = pl.pallas_call(
    _dot_kernel,
    out_shape=jax.ShapeDtypeStruct((32, BATCH // 32), jnp.float32),
)


@jax.jit
def kernel(users, items, edge_index, user_table, item_table):
    row = edge_index[0].astype(jnp.int32)
    col = edge_index[1].astype(jnp.int32)
    pad = N + jnp.arange(EP - E, dtype=jnp.int32) % (NP - N)
    row_p = jnp.concatenate([row, pad]).reshape(NW, NCH, CH)
    col_p = jnp.concatenate([col, pad]).reshape(NW, NCH, CH)

    e0 = jnp.concatenate([user_table, item_table], axis=0)
    e0 = jnp.pad(e0, ((0, NP - N), (0, 0)))
    zeros = jnp.zeros((NP, D), jnp.float32)

    deg_part = _deg_call(row_p)
    dinv_b, x = _prep_call(deg_part, e0)

    s = e0
    for _ in range(N_LAYERS):
        part = _prop_call(x, row_p, col_p, zeros)
        x, s = _combine_call(part, dinv_b, s)

    gidx = jnp.concatenate([users.astype(jnp.int32),
                            items.astype(jnp.int32) + N_USERS])
    rows = _bgather_call(s, gidx)
    scores = _dot_call(rows)
    return scores.reshape(BATCH)
